# Initial kernel scaffold; baseline (speedup 1.0000x reference)
import jax
import jax.numpy as jnp
from jax.experimental import pallas as pl


def kernel(beta, embed, slice_id, is_cp):
    raise NotImplementedError("write your pallas kernel here")


# speedup vs baseline: 6.2441x; 6.2441x over previous
import functools

import jax
import jax.numpy as jnp
from jax import lax
from jax.experimental import pallas as pl
from jax.experimental.pallas import tpu as pltpu

_B, _N, _D, _K = 4, 4096, 32, 64
_ATTR_W = 1.0
_REPL_W = 1.0
_TILE = 512
_NB = _N // _TILE


def _stats_body(beta_ref, emb_ref, sid_ref, cp_ref,
                pos_ref, bcep_ref, bcen_ref, attr_ref):
    beta = beta_ref[0, 0, :]
    E = emb_ref[0, :, :]
    sid = sid_ref[0, 0, :]
    labels = (cp_ref[0, 0, :] == 1).astype(jnp.float32)

    ax = jnp.abs(beta)
    log1p_term = jnp.log1p(jnp.exp(-ax))
    neg_log_p = jnp.maximum(-beta, 0.0) + log1p_term
    neg_log_np = jnp.maximum(beta, 0.0) + log1p_term
    b = pl.program_id(0)
    pos_ref[0, b] = jnp.sum(labels)
    bcep_ref[0, b] = jnp.sum(labels * neg_log_p)
    bcen_ref[0, b] = jnp.sum((1.0 - labels) * neg_log_np)

    inst2d = lax.broadcasted_iota(jnp.int32, (_N, _K), 1)
    pos2d = lax.broadcasted_iota(jnp.int32, (_N, _K), 0)
    S = (sid[:, None] == inst2d).astype(jnp.float32)
    cnt = jnp.sum(S, axis=0)
    sq = jnp.sum(E * E, axis=1)
    seg_sq = jnp.sum(S * sq[:, None], axis=0)
    sum_e = lax.dot_general(S, E, (((0,), (0,)), ((), ())),
                            preferred_element_type=jnp.float32)
    both = S * labels[:, None]
    has_cp = jnp.sum(both, axis=0) > 0.0
    first_idx = jnp.min(jnp.where(both > 0.0, pos2d, _N), axis=0)
    onehot_first = (pos2d == first_idx[None, :]).astype(jnp.float32)
    C = lax.dot_general(onehot_first, E, (((0,), (0,)), ((), ())),
                        preferred_element_type=jnp.float32)
    csq = jnp.sum(C * C, axis=1)
    cross = jnp.sum(sum_e * C, axis=1)
    attr_k = (seg_sq - 2.0 * cross + cnt * csq) / jnp.maximum(cnt, 1.0)
    attr_ref[0, b] = jnp.sum(jnp.where(has_cp, attr_k, 0.0))


def _repl_body(ei_ref, ej_ref, cpi_ref, cpj_ref, out_ref):
    b = pl.program_id(0)
    i = pl.program_id(1)
    j = pl.program_id(2)

    @pl.when((i == 0) & (j == 0))
    def _init():
        out_ref[0, b] = 0.0

    @pl.when(j >= i)
    def _compute():
        Ei = ei_ref[0, :, :]
        Ej = ej_ref[0, :, :]
        pmi = (cpi_ref[0, 0, 0, :] == 1).astype(jnp.float32)
        pmj = (cpj_ref[0, 0, 0, :] == 1).astype(jnp.float32)
        sqi = jnp.sum(Ei * Ei, axis=1)
        sqj = jnp.sum(Ej * Ej, axis=1)
        G = lax.dot_general(Ei, Ej, (((1,), (1,)), ((), ())),
                            preferred_element_type=jnp.float32)
        d2 = jnp.maximum(sqi[:, None] + sqj[None, :] - 2.0 * G, 0.0)
        s = jnp.sum(jnp.exp(-d2) * (pmi[:, None] * pmj[None, :]))
        factor = jnp.where(i == j, 1.0, 2.0)
        out_ref[0, b] += s * factor


def kernel(beta, embed, slice_id, is_cp):
    beta2 = beta[..., 0].reshape(_B, 1, _N)
    sid = slice_id.astype(jnp.int32).reshape(_B, 1, _N)
    cp = is_cp.astype(jnp.int32).reshape(_B, 1, _N)
    cp4 = is_cp.astype(jnp.int32).reshape(_B, _NB, 1, _TILE)

    scalar_out = jax.ShapeDtypeStruct((1, _B), jnp.float32)
    scalar_spec = pl.BlockSpec((1, _B), lambda b: (0, 0), memory_space=pltpu.SMEM)

    pos_count, bce_pos, bce_neg, attr = pl.pallas_call(
        _stats_body,
        grid=(_B,),
        in_specs=[
            pl.BlockSpec((1, 1, _N), lambda b: (b, 0, 0)),
            pl.BlockSpec((1, _N, _D), lambda b: (b, 0, 0)),
            pl.BlockSpec((1, 1, _N), lambda b: (b, 0, 0)),
            pl.BlockSpec((1, 1, _N), lambda b: (b, 0, 0)),
        ],
        out_specs=[scalar_spec] * 4,
        out_shape=[scalar_out] * 4,
    )(beta2, embed, sid, cp)
    pos_count, bce_pos, bce_neg, attr = (
        pos_count[0], bce_pos[0], bce_neg[0], attr[0])

    repl_spec = pl.BlockSpec((1, _B), lambda b, i, j: (0, 0),
                             memory_space=pltpu.SMEM)
    repl_sum = pl.pallas_call(
        _repl_body,
        grid=(_B, _NB, _NB),
        in_specs=[
            pl.BlockSpec((1, _TILE, _D), lambda b, i, j: (b, i, 0)),
            pl.BlockSpec((1, _TILE, _D), lambda b, i, j: (b, j, 0)),
            pl.BlockSpec((1, 1, 1, _TILE), lambda b, i, j: (b, i, 0, 0)),
            pl.BlockSpec((1, 1, 1, _TILE), lambda b, i, j: (b, j, 0, 0)),
        ],
        out_specs=repl_spec,
        out_shape=jax.ShapeDtypeStruct((1, _B), jnp.float32),
    )(embed, embed, cp4, cp4)
    repl_sum = repl_sum[0]

    nf = jnp.float32(_N)
    neg_count = nf - pos_count
    valid = (pos_count >= 1.0) & (neg_count >= 1.0)
    vf = valid.astype(jnp.float32)
    pos_weight = neg_count / (pos_count + 1e-06)
    beta_loss = (pos_weight * bce_pos + bce_neg) / nf
    attr_l = attr * _ATTR_W
    n_pairs = jnp.maximum(pos_count * pos_count, 1.0)
    repl_raw = repl_sum / n_pairs * _REPL_W
    repl_l = jnp.where(pos_count > 1.0, repl_raw, 0.0)

    total = jnp.sum(vf * (beta_loss + attr_l + repl_l))
    denom = jnp.maximum(jnp.sum(vf), 1.0)
    return (total / denom,
            jnp.sum(vf * beta_loss) / denom,
            jnp.sum(vf * attr_l) / denom,
            jnp.sum(vf * repl_l) / denom)
